# Initial kernel scaffold; baseline (speedup 1.0000x reference)
import jax
import jax.numpy as jnp
from jax.experimental import pallas as pl


def kernel(x, edge_index, edge_weight, W1, b1, W2, b2, Wfc, bfc):
    raise NotImplementedError("write your pallas kernel here")


# speedup vs baseline: 12.1786x; 12.1786x over previous
import functools

import jax
import jax.numpy as jnp
from jax import lax
from jax.experimental import pallas as pl
from jax.experimental.pallas import tpu as pltpu
from jax.experimental.pallas import tpu_sc as plsc

N_NODES = 100000
N_EDGES = 6400000
NUM_GRAPHS = N_NODES // 8

_NC = 2
_NS = 16
_NW = _NC * _NS

NP = 100096
EPW = N_EDGES // _NW
BLK = 128
NBLK = N_EDGES // BLK
RPT = NP // _NS
ZR = 1564
CHD = 2000
NCHD = EPW // CHD

_f32 = jnp.float32
_i32 = jnp.int32


def _deg_body(c_hbm, w_hbm, out_hbm, cbuf, wbuf, degp):
    cid = lax.axis_index("c")
    sid = lax.axis_index("s")
    wid = sid * _NC + cid

    def zero(i, _):
        degp[pl.ds(i * 16, 16)] = jnp.zeros((16,), _f32)
        return 0
    lax.fori_loop(0, NP // 16, zero, 0)

    base = pl.multiple_of(wid * EPW, 8)

    def chunk(k, _):
        off = pl.multiple_of(base + k * CHD, 8)
        pltpu.sync_copy(c_hbm.at[pl.ds(off, CHD)], cbuf)
        pltpu.sync_copy(w_hbm.at[pl.ds(off, CHD)], wbuf)

        def inner(i, _):
            idx = cbuf[pl.ds(i * 16, 16)]
            val = wbuf[pl.ds(i * 16, 16)]
            plsc.addupdate_scatter(degp, [idx], val)
            return 0
        lax.fori_loop(0, CHD // 16, inner, 0)
        return 0
    lax.fori_loop(0, NCHD, chunk, 0)

    pltpu.sync_copy(degp, out_hbm.at[pl.ds(pl.multiple_of(wid * NP, 128), NP)])


def _deg_call(c, w):
    mesh = plsc.VectorSubcoreMesh(core_axis_name="c", subcore_axis_name="s")
    return pl.kernel(
        _deg_body,
        out_type=jax.ShapeDtypeStruct((_NW * NP,), _f32),
        mesh=mesh,
        compiler_params=pltpu.CompilerParams(
            needs_layout_passes=False, use_tc_tiling_on_sc=False),
        scratch_types=[
            pltpu.VMEM((CHD,), _i32),
            pltpu.VMEM((CHD,), _f32),
            pltpu.VMEM((NP,), _f32),
        ],
    )(c, w)


def _agg_body(feat_split, r_hbm, c2d, w_hbm, p_hbm, out_hbm,
              ridx, cidx, wbuf, rows, zbuf, acc, gsem):
    cid = lax.axis_index("c")
    sid = lax.axis_index("s")

    iota = lax.iota(_i32, 16)
    rowpat = (iota >= 8).astype(_i32)
    colpat = iota - 8 * rowpat
    zero16 = jnp.zeros((16,), _f32)

    def zzero(i, _):
        plsc.store_scatter(zbuf, [rowpat + 2 * i, colpat], zero16)
        return 0
    lax.fori_loop(0, ZR // 2, zzero, 0)

    def zcopy(t, _):
        pltpu.sync_copy(zbuf, acc.at[pl.ds(sid * RPT + t * ZR, ZR)])
        return 0
    lax.fori_loop(0, RPT // ZR, zcopy, 0)
    plsc.subcore_barrier()

    if feat_split:
        bpw = NBLK // _NS
        bstart = sid * bpw
        nblocks = bpw
        idx_off = cid * N_NODES
    else:
        wid = sid * _NC + cid
        base_n = NBLK // _NW
        bstart = wid * base_n + jnp.minimum(wid, NBLK % _NW)
        nblocks = base_n + (wid < NBLK % _NW).astype(_i32)
        idx_off = 0

    def blk(b, _):
        bi = bstart + b
        eoff = pl.multiple_of(bi * BLK, 8)
        pltpu.sync_copy(r_hbm.at[pl.ds(eoff, BLK)], ridx)
        pltpu.sync_copy(c2d.at[pl.ds(bi, 1)], cidx)
        pltpu.sync_copy(w_hbm.at[pl.ds(eoff, BLK)], wbuf)

        if feat_split:
            def shift(i, _):
                ridx[pl.ds(i * 16, 16)] = ridx[pl.ds(i * 16, 16)] + idx_off
                return 0
            lax.fori_loop(0, BLK // 16, shift, 0)

        pltpu.async_copy(p_hbm.at[ridx], rows, gsem).wait()

        def scale(l2, _):
            ir = rowpat + 2 * l2
            v = plsc.load_gather(rows, [ir, colpat])
            wv = plsc.load_gather(wbuf, [ir])
            plsc.store_scatter(rows, [ir, colpat], v * wv)
            return 0
        lax.fori_loop(0, BLK // 2, scale, 0)

        pltpu.sync_copy(rows, acc.at[cidx.at[0]], add=True)
        return 0
    lax.fori_loop(0, nblocks, blk, 0)

    plsc.subcore_barrier()
    pltpu.sync_copy(acc.at[pl.ds(sid * RPT, RPT)],
                    out_hbm.at[cid, pl.ds(sid * RPT, RPT)])


def _agg_call(feat_split, r, c2d, w, p):
    mesh = plsc.VectorSubcoreMesh(core_axis_name="c", subcore_axis_name="s")
    return pl.kernel(
        functools.partial(_agg_body, feat_split),
        out_type=jax.ShapeDtypeStruct((_NC, NP, 8), _f32),
        mesh=mesh,
        compiler_params=pltpu.CompilerParams(
            needs_layout_passes=False, use_tc_tiling_on_sc=False),
        scratch_types=[
            pltpu.VMEM((BLK,), _i32),
            pltpu.VMEM((1, BLK), _i32),
            pltpu.VMEM((BLK,), _f32),
            pltpu.VMEM((BLK, 8), _f32),
            pltpu.VMEM((ZR, 8), _f32),
            pltpu.VMEM_SHARED((NP, 8), _f32),
            pltpu.SemaphoreType.DMA,
        ],
    )(r, c2d, w, p)


_BN = 2048


def _dense1_body(degp_ref, x_ref, dinv_ref, p1_ref):
    deg = jnp.sum(degp_ref[...], axis=0) + 1.0
    dinv = lax.rsqrt(deg)
    dinv_ref[...] = dinv
    p1_ref[...] = x_ref[...] * dinv[:, None]


def _dense1_call(degp, x):
    grid = pl.cdiv(N_NODES, _BN)
    return pl.pallas_call(
        _dense1_body,
        grid=(grid,),
        in_specs=[
            pl.BlockSpec((_NW, _BN), lambda i: (0, i)),
            pl.BlockSpec((_BN, 8), lambda i: (i, 0)),
        ],
        out_specs=[
            pl.BlockSpec((_BN,), lambda i: (i,)),
            pl.BlockSpec((_BN, 8), lambda i: (i, 0)),
        ],
        out_shape=[
            jax.ShapeDtypeStruct((N_NODES,), _f32),
            jax.ShapeDtypeStruct((N_NODES, 8), _f32),
        ],
    )(degp, x)


def _dense2_body(a0_ref, a1_ref, p1_ref, dinv_ref, w_ref, b_ref,
                 p2lo_ref, p2hi_ref):
    dv = dinv_ref[...][:, None]
    t = dv * (a0_ref[...] + a1_ref[...] + p1_ref[...])
    z = jnp.dot(t, w_ref[...], preferred_element_type=_f32) + b_ref[...][None, :]
    p2 = dv * jnp.maximum(z, 0.0)
    p2lo_ref[...] = p2[:, :8]
    p2hi_ref[...] = p2[:, 8:]


def _dense2_call(a0, a1, p1, dinv, w, b):
    grid = pl.cdiv(N_NODES, _BN)
    return pl.pallas_call(
        _dense2_body,
        grid=(grid,),
        in_specs=[
            pl.BlockSpec((_BN, 8), lambda i: (i, 0)),
            pl.BlockSpec((_BN, 8), lambda i: (i, 0)),
            pl.BlockSpec((_BN, 8), lambda i: (i, 0)),
            pl.BlockSpec((_BN,), lambda i: (i,)),
            pl.BlockSpec((8, 16), lambda i: (0, 0)),
            pl.BlockSpec((16,), lambda i: (0,)),
        ],
        out_specs=[
            pl.BlockSpec((_BN, 8), lambda i: (i, 0)),
            pl.BlockSpec((_BN, 8), lambda i: (i, 0)),
        ],
        out_shape=[
            jax.ShapeDtypeStruct((N_NODES, 8), _f32),
            jax.ShapeDtypeStruct((N_NODES, 8), _f32),
        ],
    )(a0, a1, p1, dinv, w, b)


def _dense3_body(alo_ref, ahi_ref, plo_ref, phi_ref, dinv_ref, w_ref, b_ref,
                 h_ref):
    dv = dinv_ref[...][:, None]
    t = dv * jnp.concatenate(
        [alo_ref[...] + plo_ref[...], ahi_ref[...] + phi_ref[...]], axis=1)
    z = jnp.dot(t, w_ref[...], preferred_element_type=_f32) + b_ref[...][None, :]
    h_ref[...] = jnp.maximum(z, 0.0)


def _dense3_call(alo, ahi, plo, phi, dinv, w, b):
    grid = pl.cdiv(N_NODES, _BN)
    return pl.pallas_call(
        _dense3_body,
        grid=(grid,),
        in_specs=[
            pl.BlockSpec((_BN, 8), lambda i: (i, 0)),
            pl.BlockSpec((_BN, 8), lambda i: (i, 0)),
            pl.BlockSpec((_BN, 8), lambda i: (i, 0)),
            pl.BlockSpec((_BN, 8), lambda i: (i, 0)),
            pl.BlockSpec((_BN,), lambda i: (i,)),
            pl.BlockSpec((16, 32), lambda i: (0, 0)),
            pl.BlockSpec((32,), lambda i: (0,)),
        ],
        out_specs=pl.BlockSpec((_BN, 32), lambda i: (i, 0)),
        out_shape=jax.ShapeDtypeStruct((N_NODES, 32), _f32),
    )(alo, ahi, plo, phi, dinv, w, b)


_BG = 512


def _fc_body(hr_ref, w_ref, b_ref, out_ref):
    logits = jnp.dot(hr_ref[...], w_ref[...], preferred_element_type=_f32)
    logits = logits + b_ref[...][None, :]
    lane = lax.broadcasted_iota(_i32, logits.shape, 1)
    neg = jnp.where(lane < 5, logits, -1e30)
    m = jnp.max(neg, axis=1, keepdims=True)
    lse = jnp.log(jnp.sum(jnp.exp(neg - m), axis=1, keepdims=True)) + m
    out_ref[...] = logits - lse


def _fc_call(hr, wf, bf):
    grid = pl.cdiv(NUM_GRAPHS, _BG)
    return pl.pallas_call(
        _fc_body,
        grid=(grid,),
        in_specs=[
            pl.BlockSpec((_BG, 256), lambda i: (i, 0)),
            pl.BlockSpec((256, 128), lambda i: (0, 0)),
            pl.BlockSpec((128,), lambda i: (0,)),
        ],
        out_specs=pl.BlockSpec((_BG, 128), lambda i: (i, 0)),
        out_shape=jax.ShapeDtypeStruct((NUM_GRAPHS, 128), _f32),
    )(hr, wf, bf)


def kernel(x, edge_index, edge_weight, W1, b1, W2, b2, Wfc, bfc):
    r = edge_index[0].astype(_i32)
    c = edge_index[1].astype(_i32)
    w = edge_weight.astype(_f32)
    c2d = c.reshape(NBLK, BLK)

    degp = _deg_call(c, w).reshape(_NW, NP)
    dinv, p1 = _dense1_call(degp, x)

    agg1 = _agg_call(False, r, c2d, w, p1)
    p2lo, p2hi = _dense2_call(agg1[0, :N_NODES], agg1[1, :N_NODES],
                              p1, dinv, W1, b1)

    p2s = jnp.concatenate([p2lo, p2hi], axis=0)
    agg2 = _agg_call(True, r, c2d, w, p2s)
    h2 = _dense3_call(agg2[0, :N_NODES], agg2[1, :N_NODES],
                      p2lo, p2hi, dinv, W2, b2)

    hr = h2.reshape(NUM_GRAPHS, 256)
    wf = jnp.pad(Wfc, ((0, 0), (0, 128 - Wfc.shape[1])))
    bf = jnp.pad(bfc, ((0, 128 - bfc.shape[0]),))
    out = _fc_call(hr, wf, bf)
    return out[:, :5]


# speedup vs baseline: 44.4979x; 3.6538x over previous
import functools

import jax
import jax.numpy as jnp
from jax import lax
from jax.experimental import pallas as pl
from jax.experimental.pallas import tpu as pltpu
from jax.experimental.pallas import tpu_sc as plsc

N_NODES = 100000
N_EDGES = 6400000
NUM_GRAPHS = N_NODES // 8

_NC = 2
_NS = 16
_NW = _NC * _NS

NP = 100096
EPW = N_EDGES // _NW
BLK = 128
K = 20
KE = K * BLK
NBP = 50560
EP = NBP * BLK
CPW1 = NBP // (_NW * K)
CPW2 = NBP // (_NS * K)
RPT = NP // _NS
ZR = 1564
CHD = 2000
NCHD = EPW // CHD

_f32 = jnp.float32
_i32 = jnp.int32


def _deg_body(c_hbm, w_hbm, out_hbm, cbuf, wbuf, degp):
    cid = lax.axis_index("c")
    sid = lax.axis_index("s")
    wid = sid * _NC + cid

    def zero(i, _):
        degp[pl.ds(i * 16, 16)] = jnp.zeros((16,), _f32)
        return 0
    lax.fori_loop(0, NP // 16, zero, 0)

    base = pl.multiple_of(wid * EPW, 8)

    def chunk(k, _):
        off = pl.multiple_of(base + k * CHD, 8)
        pltpu.sync_copy(c_hbm.at[pl.ds(off, CHD)], cbuf)
        pltpu.sync_copy(w_hbm.at[pl.ds(off, CHD)], wbuf)

        def inner(i, _):
            idx = cbuf[pl.ds(i * 16, 16)]
            val = wbuf[pl.ds(i * 16, 16)]
            plsc.addupdate_scatter(degp, [idx], val)
            return 0
        lax.fori_loop(0, CHD // 16, inner, 0)
        return 0
    lax.fori_loop(0, NCHD, chunk, 0)

    pltpu.sync_copy(degp, out_hbm.at[pl.ds(pl.multiple_of(wid * NP, 128), NP)])


def _deg_call(c, w):
    mesh = plsc.VectorSubcoreMesh(core_axis_name="c", subcore_axis_name="s")
    return pl.kernel(
        _deg_body,
        out_type=jax.ShapeDtypeStruct((_NW * NP,), _f32),
        mesh=mesh,
        compiler_params=pltpu.CompilerParams(
            needs_layout_passes=False, use_tc_tiling_on_sc=False),
        scratch_types=[
            pltpu.VMEM((CHD,), _i32),
            pltpu.VMEM((CHD,), _f32),
            pltpu.VMEM((NP,), _f32),
        ],
    )(c, w)


def _agg_body(feat_split, r_hbm, c2d, w_hbm, p_hbm, out_hbm,
              ridx, cidx, wbuf, rows, zbuf, acc, gsem, ssem):
    cid = lax.axis_index("c")
    sid = lax.axis_index("s")

    iota = lax.iota(_i32, 16)
    rowpat = (iota >= 8).astype(_i32)
    colpat = iota - 8 * rowpat
    zero16 = jnp.zeros((16,), _f32)

    def zzero(i, _):
        plsc.store_scatter(zbuf, [rowpat + 2 * i, colpat], zero16)
        return 0
    lax.fori_loop(0, ZR // 2, zzero, 0)

    def zcopy(t, _):
        pltpu.sync_copy(zbuf, acc.at[pl.ds(sid * RPT + t * ZR, ZR)])
        return 0
    lax.fori_loop(0, RPT // ZR, zcopy, 0)
    plsc.subcore_barrier()

    if feat_split:
        chbase = sid * CPW2
        nchunks = CPW2
        idx_off = cid * N_NODES
    else:
        wid = sid * _NC + cid
        chbase = wid * CPW1
        nchunks = CPW1
        idx_off = 0

    def chunk(k, _):
        ch = chbase + k
        eoff = pl.multiple_of(ch * KE, 8)
        boff = ch * K
        pltpu.sync_copy(r_hbm.at[pl.ds(eoff, KE)], ridx)
        pltpu.sync_copy(c2d.at[pl.ds(boff, K)], cidx)
        pltpu.sync_copy(w_hbm.at[pl.ds(eoff, KE)], wbuf)

        if feat_split:
            @plsc.parallel_loop(0, KE // 16, unroll=4)
            def _shift(i):
                ridx[pl.ds(i * 16, 16)] = ridx[pl.ds(i * 16, 16)] + idx_off

        def fire(j, _):
            pltpu.async_copy(
                p_hbm.at[ridx.at[pl.ds(j * BLK, BLK)]], rows.at[j], gsem)
            return 0
        lax.fori_loop(0, K, fire, 0)

        def drain(j, _):
            pltpu.make_async_copy(
                p_hbm.at[ridx.at[pl.ds(j * BLK, BLK)]], rows.at[j],
                gsem).wait()
            return 0
        lax.fori_loop(0, K, drain, 0)

        def scale_blk(j, _):
            jj = jnp.zeros((16,), _i32) + j
            wbase = j * BLK

            @plsc.parallel_loop(0, BLK // 2, unroll=4)
            def _scale(l2):
                ir = rowpat + 2 * l2
                v = plsc.load_gather(rows, [jj, ir, colpat])
                wv = plsc.load_gather(wbuf, [wbase + ir])
                plsc.store_scatter(rows, [jj, ir, colpat], v * wv)
            return 0
        lax.fori_loop(0, K, scale_blk, 0)

        def sfire(j, _):
            pltpu.async_copy(rows.at[j], acc.at[cidx.at[j]], ssem, add=True)
            return 0
        lax.fori_loop(0, K, sfire, 0)

        def sdrain(j, _):
            pltpu.make_async_copy(rows.at[j], acc.at[cidx.at[j]], ssem).wait()
            return 0
        lax.fori_loop(0, K, sdrain, 0)
        return 0
    lax.fori_loop(0, nchunks, chunk, 0)

    plsc.subcore_barrier()
    pltpu.sync_copy(acc.at[pl.ds(sid * RPT, RPT)],
                    out_hbm.at[cid, pl.ds(sid * RPT, RPT)])


def _agg_call(feat_split, r, c2d, w, p):
    mesh = plsc.VectorSubcoreMesh(core_axis_name="c", subcore_axis_name="s")
    return pl.kernel(
        functools.partial(_agg_body, feat_split),
        out_type=jax.ShapeDtypeStruct((_NC, NP, 8), _f32),
        mesh=mesh,
        compiler_params=pltpu.CompilerParams(
            needs_layout_passes=False, use_tc_tiling_on_sc=False),
        scratch_types=[
            pltpu.VMEM((KE,), _i32),
            pltpu.VMEM((K, BLK), _i32),
            pltpu.VMEM((KE,), _f32),
            pltpu.VMEM((K, BLK, 8), _f32),
            pltpu.VMEM((ZR, 8), _f32),
            pltpu.VMEM_SHARED((NP, 8), _f32),
            pltpu.SemaphoreType.DMA,
            pltpu.SemaphoreType.DMA,
        ],
    )(r, c2d, w, p)


_BN = 2048


def _dense1_body(degp_ref, x_ref, dinv_ref, p1_ref):
    deg = jnp.sum(degp_ref[...], axis=0) + 1.0
    dinv = lax.rsqrt(deg)
    dinv_ref[...] = dinv
    p1_ref[...] = x_ref[...] * dinv[:, None]


def _dense1_call(degp, x):
    grid = pl.cdiv(N_NODES, _BN)
    return pl.pallas_call(
        _dense1_body,
        grid=(grid,),
        in_specs=[
            pl.BlockSpec((_NW, _BN), lambda i: (0, i)),
            pl.BlockSpec((_BN, 8), lambda i: (i, 0)),
        ],
        out_specs=[
            pl.BlockSpec((_BN,), lambda i: (i,)),
            pl.BlockSpec((_BN, 8), lambda i: (i, 0)),
        ],
        out_shape=[
            jax.ShapeDtypeStruct((N_NODES,), _f32),
            jax.ShapeDtypeStruct((N_NODES, 8), _f32),
        ],
    )(degp, x)


def _dense2_body(a0_ref, a1_ref, p1_ref, dinv_ref, w_ref, b_ref,
                 p2lo_ref, p2hi_ref):
    dv = dinv_ref[...][:, None]
    t = dv * (a0_ref[...] + a1_ref[...] + p1_ref[...])
    z = jnp.dot(t, w_ref[...], preferred_element_type=_f32) + b_ref[...][None, :]
    p2 = dv * jnp.maximum(z, 0.0)
    p2lo_ref[...] = p2[:, :8]
    p2hi_ref[...] = p2[:, 8:]


def _dense2_call(a0, a1, p1, dinv, w, b):
    grid = pl.cdiv(N_NODES, _BN)
    return pl.pallas_call(
        _dense2_body,
        grid=(grid,),
        in_specs=[
            pl.BlockSpec((_BN, 8), lambda i: (i, 0)),
            pl.BlockSpec((_BN, 8), lambda i: (i, 0)),
            pl.BlockSpec((_BN, 8), lambda i: (i, 0)),
            pl.BlockSpec((_BN,), lambda i: (i,)),
            pl.BlockSpec((8, 16), lambda i: (0, 0)),
            pl.BlockSpec((16,), lambda i: (0,)),
        ],
        out_specs=[
            pl.BlockSpec((_BN, 8), lambda i: (i, 0)),
            pl.BlockSpec((_BN, 8), lambda i: (i, 0)),
        ],
        out_shape=[
            jax.ShapeDtypeStruct((N_NODES, 8), _f32),
            jax.ShapeDtypeStruct((N_NODES, 8), _f32),
        ],
    )(a0, a1, p1, dinv, w, b)


def _dense3_body(alo_ref, ahi_ref, plo_ref, phi_ref, dinv_ref, w_ref, b_ref,
                 h_ref):
    dv = dinv_ref[...][:, None]
    t = dv * jnp.concatenate(
        [alo_ref[...] + plo_ref[...], ahi_ref[...] + phi_ref[...]], axis=1)
    z = jnp.dot(t, w_ref[...], preferred_element_type=_f32) + b_ref[...][None, :]
    h_ref[...] = jnp.maximum(z, 0.0)


def _dense3_call(alo, ahi, plo, phi, dinv, w, b):
    grid = pl.cdiv(N_NODES, _BN)
    return pl.pallas_call(
        _dense3_body,
        grid=(grid,),
        in_specs=[
            pl.BlockSpec((_BN, 8), lambda i: (i, 0)),
            pl.BlockSpec((_BN, 8), lambda i: (i, 0)),
            pl.BlockSpec((_BN, 8), lambda i: (i, 0)),
            pl.BlockSpec((_BN, 8), lambda i: (i, 0)),
            pl.BlockSpec((_BN,), lambda i: (i,)),
            pl.BlockSpec((16, 32), lambda i: (0, 0)),
            pl.BlockSpec((32,), lambda i: (0,)),
        ],
        out_specs=pl.BlockSpec((_BN, 32), lambda i: (i, 0)),
        out_shape=jax.ShapeDtypeStruct((N_NODES, 32), _f32),
    )(alo, ahi, plo, phi, dinv, w, b)


_BG = 512


def _fc_body(hr_ref, w_ref, b_ref, out_ref):
    logits = jnp.dot(hr_ref[...], w_ref[...], preferred_element_type=_f32)
    logits = logits + b_ref[...][None, :]
    lane = lax.broadcasted_iota(_i32, logits.shape, 1)
    neg = jnp.where(lane < 5, logits, -1e30)
    m = jnp.max(neg, axis=1, keepdims=True)
    lse = jnp.log(jnp.sum(jnp.exp(neg - m), axis=1, keepdims=True)) + m
    out_ref[...] = logits - lse


def _fc_call(hr, wf, bf):
    grid = pl.cdiv(NUM_GRAPHS, _BG)
    return pl.pallas_call(
        _fc_body,
        grid=(grid,),
        in_specs=[
            pl.BlockSpec((_BG, 256), lambda i: (i, 0)),
            pl.BlockSpec((256, 128), lambda i: (0, 0)),
            pl.BlockSpec((128,), lambda i: (0,)),
        ],
        out_specs=pl.BlockSpec((_BG, 128), lambda i: (i, 0)),
        out_shape=jax.ShapeDtypeStruct((NUM_GRAPHS, 128), _f32),
    )(hr, wf, bf)


def kernel(x, edge_index, edge_weight, W1, b1, W2, b2, Wfc, bfc):
    r = edge_index[0].astype(_i32)
    c = edge_index[1].astype(_i32)
    w = edge_weight.astype(_f32)
    rp = jnp.pad(r, (0, EP - N_EDGES))
    cp = jnp.pad(c, (0, EP - N_EDGES))
    wp = jnp.pad(w, (0, EP - N_EDGES))
    c2d = cp.reshape(NBP, BLK)

    degp = _deg_call(c, w).reshape(_NW, NP)
    dinv, p1 = _dense1_call(degp, x)

    agg1 = _agg_call(False, rp, c2d, wp, p1)
    p2lo, p2hi = _dense2_call(agg1[0, :N_NODES], agg1[1, :N_NODES],
                              p1, dinv, W1, b1)

    p2s = jnp.concatenate([p2lo, p2hi], axis=0)
    agg2 = _agg_call(True, rp, c2d, wp, p2s)
    h2 = _dense3_call(agg2[0, :N_NODES], agg2[1, :N_NODES],
                      p2lo, p2hi, dinv, W2, b2)

    hr = h2.reshape(NUM_GRAPHS, 256)
    wf = jnp.pad(Wfc, ((0, 0), (0, 128 - Wfc.shape[1])))
    bf = jnp.pad(bfc, ((0, 128 - bfc.shape[0]),))
    out = _fc_call(hr, wf, bf)
    return out[:, :5]


# speedup vs baseline: 48.9842x; 1.1008x over previous
import functools

import jax
import jax.numpy as jnp
from jax import lax
from jax.experimental import pallas as pl
from jax.experimental.pallas import tpu as pltpu
from jax.experimental.pallas import tpu_sc as plsc

N_NODES = 100000
N_EDGES = 6400000
NUM_GRAPHS = N_NODES // 8

_NC = 2
_NS = 16
_NW = _NC * _NS

NP = 100096
EPW = N_EDGES // _NW
BLK = 128
K = 10
KE = K * BLK
NBP = 51200
EP = NBP * BLK
CPW1 = NBP // (_NW * K)
CPW2 = NBP // (_NS * K)
RPT = NP // _NS
ZR = 1564
CHD = 2000
NCHD = EPW // CHD

_f32 = jnp.float32
_i32 = jnp.int32


def _deg_body(c_hbm, w_hbm, out_hbm, cbuf, wbuf, degp):
    cid = lax.axis_index("c")
    sid = lax.axis_index("s")
    wid = sid * _NC + cid

    def zero(i, _):
        degp[pl.ds(i * 16, 16)] = jnp.zeros((16,), _f32)
        return 0
    lax.fori_loop(0, NP // 16, zero, 0)

    base = pl.multiple_of(wid * EPW, 8)

    def chunk(k, _):
        off = pl.multiple_of(base + k * CHD, 8)
        pltpu.sync_copy(c_hbm.at[pl.ds(off, CHD)], cbuf)
        pltpu.sync_copy(w_hbm.at[pl.ds(off, CHD)], wbuf)

        def inner(i, _):
            idx = cbuf[pl.ds(i * 16, 16)]
            val = wbuf[pl.ds(i * 16, 16)]
            plsc.addupdate_scatter(degp, [idx], val)
            return 0
        lax.fori_loop(0, CHD // 16, inner, 0)
        return 0
    lax.fori_loop(0, NCHD, chunk, 0)

    pltpu.sync_copy(degp, out_hbm.at[pl.ds(pl.multiple_of(wid * NP, 128), NP)])


def _deg_call(c, w):
    mesh = plsc.VectorSubcoreMesh(core_axis_name="c", subcore_axis_name="s")
    return pl.kernel(
        _deg_body,
        out_type=jax.ShapeDtypeStruct((_NW * NP,), _f32),
        mesh=mesh,
        compiler_params=pltpu.CompilerParams(
            needs_layout_passes=False, use_tc_tiling_on_sc=False),
        scratch_types=[
            pltpu.VMEM((CHD,), _i32),
            pltpu.VMEM((CHD,), _f32),
            pltpu.VMEM((NP,), _f32),
        ],
    )(c, w)


def _agg_body(feat_split, r_hbm, c2d, w_hbm, p_hbm, out_hbm,
              ridx0, ridx1, cidx0, cidx1, wbuf0, wbuf1, rows0, rows1,
              zbuf, acc, gsem0, gsem1, ssem0, ssem1, stsem0, stsem1):
    cid = lax.axis_index("c")
    sid = lax.axis_index("s")

    iota = lax.iota(_i32, 16)
    rowpat = (iota >= 8).astype(_i32)
    colpat = iota - 8 * rowpat
    zero16 = jnp.zeros((16,), _f32)

    def zzero(i, _):
        plsc.store_scatter(zbuf, [rowpat + 2 * i, colpat], zero16)
        return 0
    lax.fori_loop(0, ZR // 2, zzero, 0)

    def zcopy(t, _):
        pltpu.sync_copy(zbuf, acc.at[pl.ds(sid * RPT + t * ZR, ZR)])
        return 0
    lax.fori_loop(0, RPT // ZR, zcopy, 0)
    plsc.subcore_barrier()

    if feat_split:
        chbase = sid * CPW2
        nch = CPW2
        idx_off = cid * N_NODES
    else:
        wid = sid * _NC + cid
        chbase = wid * CPW1
        nch = CPW1
        idx_off = 0
    nchh = nch // 2

    bufs = [(ridx0, cidx0, wbuf0, rows0, gsem0, ssem0, stsem0),
            (ridx1, cidx1, wbuf1, rows1, gsem1, ssem1, stsem1)]

    def fire_stage_r(c, par):
        ridx, _, _, _, _, _, stsem = bufs[par]
        eoff = pl.multiple_of((chbase + c) * KE, 8)
        pltpu.async_copy(r_hbm.at[pl.ds(eoff, KE)], ridx, stsem)

    def fire_stage_cw(c, par):
        _, cidx, wbuf, _, _, _, stsem = bufs[par]
        ch = chbase + c
        eoff = pl.multiple_of(ch * KE, 8)
        pltpu.async_copy(c2d.at[pl.ds(ch * K, K)], cidx, stsem)
        pltpu.async_copy(w_hbm.at[pl.ds(eoff, KE)], wbuf, stsem)

    def drain_stage(c, par):
        ridx, cidx, wbuf, _, _, _, stsem = bufs[par]
        ch = chbase + c
        eoff = pl.multiple_of(ch * KE, 8)
        pltpu.make_async_copy(r_hbm.at[pl.ds(eoff, KE)], ridx, stsem).wait()
        pltpu.make_async_copy(c2d.at[pl.ds(ch * K, K)], cidx, stsem).wait()
        pltpu.make_async_copy(w_hbm.at[pl.ds(eoff, KE)], wbuf, stsem).wait()

    def shift_fire_gathers(par):
        ridx, _, _, rows, gsem, _, _ = bufs[par]
        if feat_split:
            @plsc.parallel_loop(0, KE // 16, unroll=4)
            def _shift(i):
                ridx[pl.ds(i * 16, 16)] = ridx[pl.ds(i * 16, 16)] + idx_off

        def fire(j, _):
            pltpu.async_copy(
                p_hbm.at[ridx.at[pl.ds(j * BLK, BLK)]], rows.at[j], gsem)
            return 0
        lax.fori_loop(0, K, fire, 0)

    def drain_gathers(par):
        ridx, _, _, rows, gsem, _, _ = bufs[par]

        def drain(j, _):
            pltpu.make_async_copy(
                p_hbm.at[ridx.at[pl.ds(j * BLK, BLK)]], rows.at[j],
                gsem).wait()
            return 0
        lax.fori_loop(0, K, drain, 0)

    def scale_scatter(par):
        _, cidx, wbuf, rows, _, ssem, _ = bufs[par]

        def scale_blk(j, _):
            jj = jnp.zeros((16,), _i32) + j
            wbase = j * BLK

            @plsc.parallel_loop(0, BLK // 2, unroll=4)
            def _scale(l2):
                ir = rowpat + 2 * l2
                v = plsc.load_gather(rows, [jj, ir, colpat])
                wv = plsc.load_gather(wbuf, [wbase + ir])
                plsc.store_scatter(rows, [jj, ir, colpat], v * wv)

            pltpu.async_copy(rows.at[j], acc.at[cidx.at[j]], ssem, add=True)
            return 0
        lax.fori_loop(0, K, scale_blk, 0)

        def sdrain(j, _):
            pltpu.make_async_copy(rows.at[j], acc.at[cidx.at[j]], ssem).wait()
            return 0
        lax.fori_loop(0, K, sdrain, 0)

    fire_stage_r(0, 0)
    fire_stage_cw(0, 0)
    fire_stage_r(1, 1)
    fire_stage_cw(1, 1)
    drain_stage(0, 0)
    shift_fire_gathers(0)

    def pair(b2, _):
        drain_gathers(0)
        drain_stage(2 * b2 + 1, 1)
        shift_fire_gathers(1)

        @pl.when(b2 + 1 < nchh)
        def _():
            fire_stage_r(2 * b2 + 2, 0)
        scale_scatter(0)

        @pl.when(b2 + 1 < nchh)
        def _():
            fire_stage_cw(2 * b2 + 2, 0)

        drain_gathers(1)

        @pl.when(b2 + 1 < nchh)
        def _():
            drain_stage(2 * b2 + 2, 0)
            shift_fire_gathers(0)
            fire_stage_r(2 * b2 + 3, 1)
        scale_scatter(1)

        @pl.when(b2 + 1 < nchh)
        def _():
            fire_stage_cw(2 * b2 + 3, 1)
        return 0
    lax.fori_loop(0, nchh, pair, 0)

    plsc.subcore_barrier()
    pltpu.sync_copy(acc.at[pl.ds(sid * RPT, RPT)],
                    out_hbm.at[cid, pl.ds(sid * RPT, RPT)])


def _agg_call(feat_split, r, c2d, w, p):
    mesh = plsc.VectorSubcoreMesh(core_axis_name="c", subcore_axis_name="s")
    return pl.kernel(
        functools.partial(_agg_body, feat_split),
        out_type=jax.ShapeDtypeStruct((_NC, NP, 8), _f32),
        mesh=mesh,
        compiler_params=pltpu.CompilerParams(
            needs_layout_passes=False, use_tc_tiling_on_sc=False),
        scratch_types=[
            pltpu.VMEM((KE,), _i32),
            pltpu.VMEM((KE,), _i32),
            pltpu.VMEM((K, BLK), _i32),
            pltpu.VMEM((K, BLK), _i32),
            pltpu.VMEM((KE,), _f32),
            pltpu.VMEM((KE,), _f32),
            pltpu.VMEM((K, BLK, 8), _f32),
            pltpu.VMEM((K, BLK, 8), _f32),
            pltpu.VMEM((ZR, 8), _f32),
            pltpu.VMEM_SHARED((NP, 8), _f32),
            pltpu.SemaphoreType.DMA,
            pltpu.SemaphoreType.DMA,
            pltpu.SemaphoreType.DMA,
            pltpu.SemaphoreType.DMA,
            pltpu.SemaphoreType.DMA,
            pltpu.SemaphoreType.DMA,
        ],
    )(r, c2d, w, p)


_BN = 2048


def _dense1_body(degp_ref, x_ref, dinv_ref, p1_ref):
    deg = jnp.sum(degp_ref[...], axis=0) + 1.0
    dinv = lax.rsqrt(deg)
    dinv_ref[...] = dinv
    p1_ref[...] = x_ref[...] * dinv[:, None]


def _dense1_call(degp, x):
    grid = pl.cdiv(N_NODES, _BN)
    return pl.pallas_call(
        _dense1_body,
        grid=(grid,),
        in_specs=[
            pl.BlockSpec((_NW, _BN), lambda i: (0, i)),
            pl.BlockSpec((_BN, 8), lambda i: (i, 0)),
        ],
        out_specs=[
            pl.BlockSpec((_BN,), lambda i: (i,)),
            pl.BlockSpec((_BN, 8), lambda i: (i, 0)),
        ],
        out_shape=[
            jax.ShapeDtypeStruct((N_NODES,), _f32),
            jax.ShapeDtypeStruct((N_NODES, 8), _f32),
        ],
    )(degp, x)


def _dense2_body(a0_ref, a1_ref, p1_ref, dinv_ref, w_ref, b_ref,
                 p2lo_ref, p2hi_ref):
    dv = dinv_ref[...][:, None]
    t = dv * (a0_ref[...] + a1_ref[...] + p1_ref[...])
    z = jnp.dot(t, w_ref[...], preferred_element_type=_f32) + b_ref[...][None, :]
    p2 = dv * jnp.maximum(z, 0.0)
    p2lo_ref[...] = p2[:, :8]
    p2hi_ref[...] = p2[:, 8:]


def _dense2_call(a0, a1, p1, dinv, w, b):
    grid = pl.cdiv(N_NODES, _BN)
    return pl.pallas_call(
        _dense2_body,
        grid=(grid,),
        in_specs=[
            pl.BlockSpec((_BN, 8), lambda i: (i, 0)),
            pl.BlockSpec((_BN, 8), lambda i: (i, 0)),
            pl.BlockSpec((_BN, 8), lambda i: (i, 0)),
            pl.BlockSpec((_BN,), lambda i: (i,)),
            pl.BlockSpec((8, 16), lambda i: (0, 0)),
            pl.BlockSpec((16,), lambda i: (0,)),
        ],
        out_specs=[
            pl.BlockSpec((_BN, 8), lambda i: (i, 0)),
            pl.BlockSpec((_BN, 8), lambda i: (i, 0)),
        ],
        out_shape=[
            jax.ShapeDtypeStruct((N_NODES, 8), _f32),
            jax.ShapeDtypeStruct((N_NODES, 8), _f32),
        ],
    )(a0, a1, p1, dinv, w, b)


def _dense3_body(alo_ref, ahi_ref, plo_ref, phi_ref, dinv_ref, w_ref, b_ref,
                 h_ref):
    dv = dinv_ref[...][:, None]
    t = dv * jnp.concatenate(
        [alo_ref[...] + plo_ref[...], ahi_ref[...] + phi_ref[...]], axis=1)
    z = jnp.dot(t, w_ref[...], preferred_element_type=_f32) + b_ref[...][None, :]
    h_ref[...] = jnp.maximum(z, 0.0)


def _dense3_call(alo, ahi, plo, phi, dinv, w, b):
    grid = pl.cdiv(N_NODES, _BN)
    return pl.pallas_call(
        _dense3_body,
        grid=(grid,),
        in_specs=[
            pl.BlockSpec((_BN, 8), lambda i: (i, 0)),
            pl.BlockSpec((_BN, 8), lambda i: (i, 0)),
            pl.BlockSpec((_BN, 8), lambda i: (i, 0)),
            pl.BlockSpec((_BN, 8), lambda i: (i, 0)),
            pl.BlockSpec((_BN,), lambda i: (i,)),
            pl.BlockSpec((16, 32), lambda i: (0, 0)),
            pl.BlockSpec((32,), lambda i: (0,)),
        ],
        out_specs=pl.BlockSpec((_BN, 32), lambda i: (i, 0)),
        out_shape=jax.ShapeDtypeStruct((N_NODES, 32), _f32),
    )(alo, ahi, plo, phi, dinv, w, b)


_BG = 512


def _fc_body(hr_ref, w_ref, b_ref, out_ref):
    logits = jnp.dot(hr_ref[...], w_ref[...], preferred_element_type=_f32)
    logits = logits + b_ref[...][None, :]
    lane = lax.broadcasted_iota(_i32, logits.shape, 1)
    neg = jnp.where(lane < 5, logits, -1e30)
    m = jnp.max(neg, axis=1, keepdims=True)
    lse = jnp.log(jnp.sum(jnp.exp(neg - m), axis=1, keepdims=True)) + m
    out_ref[...] = logits - lse


def _fc_call(hr, wf, bf):
    grid = pl.cdiv(NUM_GRAPHS, _BG)
    return pl.pallas_call(
        _fc_body,
        grid=(grid,),
        in_specs=[
            pl.BlockSpec((_BG, 256), lambda i: (i, 0)),
            pl.BlockSpec((256, 128), lambda i: (0, 0)),
            pl.BlockSpec((128,), lambda i: (0,)),
        ],
        out_specs=pl.BlockSpec((_BG, 128), lambda i: (i, 0)),
        out_shape=jax.ShapeDtypeStruct((NUM_GRAPHS, 128), _f32),
    )(hr, wf, bf)


def kernel(x, edge_index, edge_weight, W1, b1, W2, b2, Wfc, bfc):
    r = edge_index[0].astype(_i32)
    c = edge_index[1].astype(_i32)
    w = edge_weight.astype(_f32)
    rp = jnp.pad(r, (0, EP - N_EDGES))
    cp = jnp.pad(c, (0, EP - N_EDGES))
    wp = jnp.pad(w, (0, EP - N_EDGES))
    c2d = cp.reshape(NBP, BLK)

    degp = _deg_call(c, w).reshape(_NW, NP)
    dinv, p1 = _dense1_call(degp, x)

    agg1 = _agg_call(False, rp, c2d, wp, p1)
    p2lo, p2hi = _dense2_call(agg1[0, :N_NODES], agg1[1, :N_NODES],
                              p1, dinv, W1, b1)

    p2s = jnp.concatenate([p2lo, p2hi], axis=0)
    agg2 = _agg_call(True, rp, c2d, wp, p2s)
    h2 = _dense3_call(agg2[0, :N_NODES], agg2[1, :N_NODES],
                      p2lo, p2hi, dinv, W2, b2)

    hr = h2.reshape(NUM_GRAPHS, 256)
    wf = jnp.pad(Wfc, ((0, 0), (0, 128 - Wfc.shape[1])))
    bf = jnp.pad(bfc, ((0, 128 - bfc.shape[0]),))
    out = _fc_call(hr, wf, bf)
    return out[:, :5]


# speedup vs baseline: 48.9934x; 1.0002x over previous
import functools

import jax
import jax.numpy as jnp
from jax import lax
from jax.experimental import pallas as pl
from jax.experimental.pallas import tpu as pltpu
from jax.experimental.pallas import tpu_sc as plsc

N_NODES = 100000
N_EDGES = 6400000
NUM_GRAPHS = N_NODES // 8

_NC = 2
_NS = 16
_NW = _NC * _NS

NP = 100096
EPW = N_EDGES // _NW
BLK = 128
K = 10
KE = K * BLK
NBP = 51200
EP = NBP * BLK
CPW1 = NBP // (_NW * K)
CPW2 = NBP // (_NS * K)
RPT = NP // _NS
ZR = 1564
CHD = 2000
NCHD = EPW // CHD

_f32 = jnp.float32
_i32 = jnp.int32


def _deg_body(c_hbm, w_hbm, out_hbm, cbuf, wbuf, degp):
    cid = lax.axis_index("c")
    sid = lax.axis_index("s")
    wid = sid * _NC + cid

    def zero(i, _):
        degp[pl.ds(i * 16, 16)] = jnp.zeros((16,), _f32)
        return 0
    lax.fori_loop(0, NP // 16, zero, 0)

    base = pl.multiple_of(wid * EPW, 8)

    def chunk(k, _):
        off = pl.multiple_of(base + k * CHD, 8)
        pltpu.sync_copy(c_hbm.at[pl.ds(off, CHD)], cbuf)
        pltpu.sync_copy(w_hbm.at[pl.ds(off, CHD)], wbuf)

        def inner(i, _):
            idx = cbuf[pl.ds(i * 16, 16)]
            val = wbuf[pl.ds(i * 16, 16)]
            plsc.addupdate_scatter(degp, [idx], val)
            return 0
        lax.fori_loop(0, CHD // 16, inner, 0)
        return 0
    lax.fori_loop(0, NCHD, chunk, 0)

    pltpu.sync_copy(degp, out_hbm.at[pl.ds(pl.multiple_of(wid * NP, 128), NP)])


def _deg_call(c, w):
    mesh = plsc.VectorSubcoreMesh(core_axis_name="c", subcore_axis_name="s")
    return pl.kernel(
        _deg_body,
        out_type=jax.ShapeDtypeStruct((_NW * NP,), _f32),
        mesh=mesh,
        compiler_params=pltpu.CompilerParams(
            needs_layout_passes=False, use_tc_tiling_on_sc=False),
        scratch_types=[
            pltpu.VMEM((CHD,), _i32),
            pltpu.VMEM((CHD,), _f32),
            pltpu.VMEM((NP,), _f32),
        ],
    )(c, w)


def _agg_body(feat_split, r_hbm, c2d, w_hbm, p_hbm, out_hbm,
              ridx0, ridx1, cidx0, cidx1, wbuf0, wbuf1, rows0, rows1,
              zbuf, acc, gsem0, gsem1, ssem0, ssem1, stsem0, stsem1):
    cid = lax.axis_index("c")
    sid = lax.axis_index("s")

    iota = lax.iota(_i32, 16)
    rowpat = (iota >= 8).astype(_i32)
    colpat = iota - 8 * rowpat
    zero16 = jnp.zeros((16,), _f32)

    def zzero(i, _):
        plsc.store_scatter(zbuf, [rowpat + 2 * i, colpat], zero16)
        return 0
    lax.fori_loop(0, ZR // 2, zzero, 0)

    def zcopy(t, _):
        pltpu.sync_copy(zbuf, acc.at[pl.ds(sid * RPT + t * ZR, ZR)])
        return 0
    lax.fori_loop(0, RPT // ZR, zcopy, 0)
    plsc.subcore_barrier()

    if feat_split:
        chbase = sid * CPW2
        nch = CPW2
        idx_off = cid * N_NODES
    else:
        wid = cid * _NS + sid
        chbase = wid * CPW1
        nch = CPW1
        idx_off = 0
    nchh = nch // 2

    bufs = [(ridx0, cidx0, wbuf0, rows0, gsem0, ssem0, stsem0),
            (ridx1, cidx1, wbuf1, rows1, gsem1, ssem1, stsem1)]

    def fire_stage_r(c, par):
        ridx, _, _, _, _, _, stsem = bufs[par]
        eoff = pl.multiple_of((chbase + c) * KE, 8)
        pltpu.async_copy(r_hbm.at[pl.ds(eoff, KE)], ridx, stsem)

    def fire_stage_cw(c, par):
        _, cidx, wbuf, _, _, _, stsem = bufs[par]
        ch = chbase + c
        eoff = pl.multiple_of(ch * KE, 8)
        pltpu.async_copy(c2d.at[pl.ds(ch * K, K)], cidx, stsem)
        pltpu.async_copy(w_hbm.at[pl.ds(eoff, KE)], wbuf, stsem)

    def drain_stage(c, par):
        ridx, cidx, wbuf, _, _, _, stsem = bufs[par]
        ch = chbase + c
        eoff = pl.multiple_of(ch * KE, 8)
        pltpu.make_async_copy(r_hbm.at[pl.ds(eoff, KE)], ridx, stsem).wait()
        pltpu.make_async_copy(c2d.at[pl.ds(ch * K, K)], cidx, stsem).wait()
        pltpu.make_async_copy(w_hbm.at[pl.ds(eoff, KE)], wbuf, stsem).wait()

    def shift_fire_gathers(par):
        ridx, _, _, rows, gsem, _, _ = bufs[par]
        if feat_split:
            @plsc.parallel_loop(0, KE // 16, unroll=4)
            def _shift(i):
                ridx[pl.ds(i * 16, 16)] = ridx[pl.ds(i * 16, 16)] + idx_off

        def fire(j, _):
            pltpu.async_copy(
                p_hbm.at[ridx.at[pl.ds(j * BLK, BLK)]], rows.at[j], gsem)
            return 0
        lax.fori_loop(0, K, fire, 0)

    def drain_gathers(par):
        ridx, _, _, rows, gsem, _, _ = bufs[par]

        def drain(j, _):
            pltpu.make_async_copy(
                p_hbm.at[ridx.at[pl.ds(j * BLK, BLK)]], rows.at[j],
                gsem).wait()
            return 0
        lax.fori_loop(0, K, drain, 0)

    def scale_scatter(par):
        _, cidx, wbuf, rows, _, ssem, _ = bufs[par]

        def scale_blk(j, _):
            jj = jnp.zeros((16,), _i32) + j
            wbase = j * BLK

            @plsc.parallel_loop(0, BLK // 2, unroll=4)
            def _scale(l2):
                ir = rowpat + 2 * l2
                v = plsc.load_gather(rows, [jj, ir, colpat])
                wv = plsc.load_gather(wbuf, [wbase + ir])
                plsc.store_scatter(rows, [jj, ir, colpat], v * wv)

            pltpu.async_copy(rows.at[j], acc.at[cidx.at[j]], ssem, add=True)
            return 0
        lax.fori_loop(0, K, scale_blk, 0)

        def sdrain(j, _):
            pltpu.make_async_copy(rows.at[j], acc.at[cidx.at[j]], ssem).wait()
            return 0
        lax.fori_loop(0, K, sdrain, 0)

    fire_stage_r(0, 0)
    fire_stage_cw(0, 0)
    fire_stage_r(1, 1)
    fire_stage_cw(1, 1)
    drain_stage(0, 0)
    shift_fire_gathers(0)

    def pair(b2, _):
        drain_gathers(0)
        drain_stage(2 * b2 + 1, 1)
        shift_fire_gathers(1)

        @pl.when(b2 + 1 < nchh)
        def _():
            fire_stage_r(2 * b2 + 2, 0)
        scale_scatter(0)

        @pl.when(b2 + 1 < nchh)
        def _():
            fire_stage_cw(2 * b2 + 2, 0)

        drain_gathers(1)

        @pl.when(b2 + 1 < nchh)
        def _():
            drain_stage(2 * b2 + 2, 0)
            shift_fire_gathers(0)
            fire_stage_r(2 * b2 + 3, 1)
        scale_scatter(1)

        @pl.when(b2 + 1 < nchh)
        def _():
            fire_stage_cw(2 * b2 + 3, 1)
        return 0
    lax.fori_loop(0, nchh, pair, 0)

    plsc.subcore_barrier()
    pltpu.sync_copy(acc.at[pl.ds(sid * RPT, RPT)],
                    out_hbm.at[cid, pl.ds(sid * RPT, RPT)])


def _agg_call(feat_split, r, c2d, w, p):
    mesh = plsc.VectorSubcoreMesh(core_axis_name="c", subcore_axis_name="s")
    return pl.kernel(
        functools.partial(_agg_body, feat_split),
        out_type=jax.ShapeDtypeStruct((_NC, NP, 8), _f32),
        mesh=mesh,
        compiler_params=pltpu.CompilerParams(
            needs_layout_passes=False, use_tc_tiling_on_sc=False),
        scratch_types=[
            pltpu.VMEM((KE,), _i32),
            pltpu.VMEM((KE,), _i32),
            pltpu.VMEM((K, BLK), _i32),
            pltpu.VMEM((K, BLK), _i32),
            pltpu.VMEM((KE,), _f32),
            pltpu.VMEM((KE,), _f32),
            pltpu.VMEM((K, BLK, 8), _f32),
            pltpu.VMEM((K, BLK, 8), _f32),
            pltpu.VMEM((ZR, 8), _f32),
            pltpu.VMEM_SHARED((NP, 8), _f32),
            pltpu.SemaphoreType.DMA,
            pltpu.SemaphoreType.DMA,
            pltpu.SemaphoreType.DMA,
            pltpu.SemaphoreType.DMA,
            pltpu.SemaphoreType.DMA,
            pltpu.SemaphoreType.DMA,
        ],
    )(r, c2d, w, p)


_BN = 2048


def _dense1_body(degp_ref, x_ref, dinv_ref, p1_ref):
    deg = jnp.sum(degp_ref[...], axis=0) + 1.0
    dinv = lax.rsqrt(deg)
    dinv_ref[...] = dinv
    p1_ref[...] = x_ref[...] * dinv[:, None]


def _dense1_call(degp, x):
    grid = pl.cdiv(N_NODES, _BN)
    return pl.pallas_call(
        _dense1_body,
        grid=(grid,),
        in_specs=[
            pl.BlockSpec((_NW, _BN), lambda i: (0, i)),
            pl.BlockSpec((_BN, 8), lambda i: (i, 0)),
        ],
        out_specs=[
            pl.BlockSpec((_BN,), lambda i: (i,)),
            pl.BlockSpec((_BN, 8), lambda i: (i, 0)),
        ],
        out_shape=[
            jax.ShapeDtypeStruct((N_NODES,), _f32),
            jax.ShapeDtypeStruct((N_NODES, 8), _f32),
        ],
    )(degp, x)


def _dense2_body(a0_ref, a1_ref, p1_ref, dinv_ref, w_ref, b_ref,
                 p2lo_ref, p2hi_ref):
    dv = dinv_ref[...][:, None]
    t = dv * (a0_ref[...] + a1_ref[...] + p1_ref[...])
    z = jnp.dot(t, w_ref[...], preferred_element_type=_f32) + b_ref[...][None, :]
    p2 = dv * jnp.maximum(z, 0.0)
    p2lo_ref[...] = p2[:, :8]
    p2hi_ref[...] = p2[:, 8:]


def _dense2_call(a0, a1, p1, dinv, w, b):
    grid = pl.cdiv(N_NODES, _BN)
    return pl.pallas_call(
        _dense2_body,
        grid=(grid,),
        in_specs=[
            pl.BlockSpec((_BN, 8), lambda i: (i, 0)),
            pl.BlockSpec((_BN, 8), lambda i: (i, 0)),
            pl.BlockSpec((_BN, 8), lambda i: (i, 0)),
            pl.BlockSpec((_BN,), lambda i: (i,)),
            pl.BlockSpec((8, 16), lambda i: (0, 0)),
            pl.BlockSpec((16,), lambda i: (0,)),
        ],
        out_specs=[
            pl.BlockSpec((_BN, 8), lambda i: (i, 0)),
            pl.BlockSpec((_BN, 8), lambda i: (i, 0)),
        ],
        out_shape=[
            jax.ShapeDtypeStruct((N_NODES, 8), _f32),
            jax.ShapeDtypeStruct((N_NODES, 8), _f32),
        ],
    )(a0, a1, p1, dinv, w, b)


def _dense3_body(alo_ref, ahi_ref, plo_ref, phi_ref, dinv_ref, w_ref, b_ref,
                 h_ref):
    dv = dinv_ref[...][:, None]
    t = dv * jnp.concatenate(
        [alo_ref[...] + plo_ref[...], ahi_ref[...] + phi_ref[...]], axis=1)
    z = jnp.dot(t, w_ref[...], preferred_element_type=_f32) + b_ref[...][None, :]
    h_ref[...] = jnp.maximum(z, 0.0)


def _dense3_call(alo, ahi, plo, phi, dinv, w, b):
    grid = pl.cdiv(N_NODES, _BN)
    return pl.pallas_call(
        _dense3_body,
        grid=(grid,),
        in_specs=[
            pl.BlockSpec((_BN, 8), lambda i: (i, 0)),
            pl.BlockSpec((_BN, 8), lambda i: (i, 0)),
            pl.BlockSpec((_BN, 8), lambda i: (i, 0)),
            pl.BlockSpec((_BN, 8), lambda i: (i, 0)),
            pl.BlockSpec((_BN,), lambda i: (i,)),
            pl.BlockSpec((16, 32), lambda i: (0, 0)),
            pl.BlockSpec((32,), lambda i: (0,)),
        ],
        out_specs=pl.BlockSpec((_BN, 32), lambda i: (i, 0)),
        out_shape=jax.ShapeDtypeStruct((N_NODES, 32), _f32),
    )(alo, ahi, plo, phi, dinv, w, b)


_BG = 512


def _fc_body(hr_ref, w_ref, b_ref, out_ref):
    logits = jnp.dot(hr_ref[...], w_ref[...], preferred_element_type=_f32)
    logits = logits + b_ref[...][None, :]
    lane = lax.broadcasted_iota(_i32, logits.shape, 1)
    neg = jnp.where(lane < 5, logits, -1e30)
    m = jnp.max(neg, axis=1, keepdims=True)
    lse = jnp.log(jnp.sum(jnp.exp(neg - m), axis=1, keepdims=True)) + m
    out_ref[...] = logits - lse


def _fc_call(hr, wf, bf):
    grid = pl.cdiv(NUM_GRAPHS, _BG)
    return pl.pallas_call(
        _fc_body,
        grid=(grid,),
        in_specs=[
            pl.BlockSpec((_BG, 256), lambda i: (i, 0)),
            pl.BlockSpec((256, 128), lambda i: (0, 0)),
            pl.BlockSpec((128,), lambda i: (0,)),
        ],
        out_specs=pl.BlockSpec((_BG, 128), lambda i: (i, 0)),
        out_shape=jax.ShapeDtypeStruct((NUM_GRAPHS, 128), _f32),
    )(hr, wf, bf)


def kernel(x, edge_index, edge_weight, W1, b1, W2, b2, Wfc, bfc):
    r = edge_index[0].astype(_i32)
    c = edge_index[1].astype(_i32)
    w = edge_weight.astype(_f32)
    rp = jnp.pad(r, (0, EP - N_EDGES))
    cp = jnp.pad(c, (0, EP - N_EDGES))
    wp = jnp.pad(w, (0, EP - N_EDGES))
    c2d = cp.reshape(NBP, BLK)

    degp = _deg_call(c, w).reshape(_NW, NP)
    dinv, p1 = _dense1_call(degp, x)

    agg1 = _agg_call(False, rp, c2d, wp, p1)
    p2lo, p2hi = _dense2_call(agg1[0], agg1[1], p1, dinv, W1, b1)

    p2s = jnp.concatenate([p2lo, p2hi], axis=0)
    agg2 = _agg_call(True, rp, c2d, wp, p2s)
    h2 = _dense3_call(agg2[0], agg2[1], p2lo, p2hi, dinv, W2, b2)

    hr = h2.reshape(NUM_GRAPHS, 256)
    wf = jnp.pad(Wfc, ((0, 0), (0, 128 - Wfc.shape[1])))
    bf = jnp.pad(bfc, ((0, 128 - bfc.shape[0]),))
    out = _fc_call(hr, wf, bf)
    return out[:, :5]


# speedup vs baseline: 76.9123x; 1.5699x over previous
import functools

import jax
import jax.numpy as jnp
from jax import lax
from jax.experimental import pallas as pl
from jax.experimental.pallas import tpu as pltpu
from jax.experimental.pallas import tpu_sc as plsc

N_NODES = 100000
N_EDGES = 6400000
NUM_GRAPHS = N_NODES // 8

_NC = 2
_NS = 16
_NW = _NC * _NS

NP = 100096
EPW = N_EDGES // _NW
BLK = 128
K = 10
KE = K * BLK
NBLK = N_EDGES // BLK
NCH = NBLK // K
RPT = NP // _NS
ZR = 1564
CHD = 2000
NCHD = EPW // CHD

_f32 = jnp.float32
_i32 = jnp.int32


def _deg_body(c_hbm, w_hbm, out_hbm, cbuf, wbuf, degp):
    cid = lax.axis_index("c")
    sid = lax.axis_index("s")
    wid = sid * _NC + cid

    def zero(i, _):
        degp[pl.ds(i * 16, 16)] = jnp.zeros((16,), _f32)
        return 0
    lax.fori_loop(0, NP // 16, zero, 0)

    base = pl.multiple_of(wid * EPW, 8)

    def chunk(k, _):
        off = pl.multiple_of(base + k * CHD, 8)
        pltpu.sync_copy(c_hbm.at[pl.ds(off, CHD)], cbuf)
        pltpu.sync_copy(w_hbm.at[pl.ds(off, CHD)], wbuf)

        def inner(i, _):
            idx = cbuf[pl.ds(i * 16, 16)]
            val = wbuf[pl.ds(i * 16, 16)]
            plsc.addupdate_scatter(degp, [idx], val)
            return 0
        lax.fori_loop(0, CHD // 16, inner, 0)
        return 0
    lax.fori_loop(0, NCHD, chunk, 0)

    pltpu.sync_copy(degp, out_hbm.at[pl.ds(pl.multiple_of(wid * NP, 128), NP)])


def _deg_call(c, w):
    mesh = plsc.VectorSubcoreMesh(core_axis_name="c", subcore_axis_name="s")
    return pl.kernel(
        _deg_body,
        out_type=jax.ShapeDtypeStruct((_NW * NP,), _f32),
        mesh=mesh,
        compiler_params=pltpu.CompilerParams(
            needs_layout_passes=False, use_tc_tiling_on_sc=False),
        scratch_types=[
            pltpu.VMEM((CHD,), _i32),
            pltpu.VMEM((CHD,), _f32),
            pltpu.VMEM((NP,), _f32),
        ],
    )(c, w)


def _agg_body(feat_split, r_hbm, c2d, w_hbm, p_hbm, out_hbm,
              ridx0, ridx1, cidx0, cidx1, wbuf0, wbuf1, rows0, rows1,
              zbuf, acc, gsem0, gsem1, ssem0, ssem1, stsem0, stsem1):
    cid = lax.axis_index("c")
    sid = lax.axis_index("s")

    iota = lax.iota(_i32, 16)
    rowpat = (iota >= 8).astype(_i32)
    colpat = iota - 8 * rowpat
    zero16 = jnp.zeros((16,), _f32)

    def zzero(i, _):
        plsc.store_scatter(zbuf, [rowpat + 2 * i, colpat], zero16)
        return 0
    lax.fori_loop(0, ZR // 2, zzero, 0)

    def zcopy(t, _):
        pltpu.sync_copy(zbuf, acc.at[pl.ds(sid * RPT + t * ZR, ZR)])
        return 0
    lax.fori_loop(0, RPT // ZR, zcopy, 0)
    plsc.subcore_barrier()

    if feat_split:
        t = sid
        chbase = jnp.where(t < 4, t * 314, 4 * 314 + (t - 4) * 312)
        nchh = jnp.where(t < 4, 157, 156)
        idx_off = cid * N_NODES
    else:
        wid = cid * _NS + sid
        chbase = jnp.where(wid < 4, wid * 158, 4 * 158 + (wid - 4) * 156)
        nchh = jnp.where(wid < 4, 79, 78)
        idx_off = 0

    bufs = [(ridx0, cidx0, wbuf0, rows0, gsem0, ssem0, stsem0),
            (ridx1, cidx1, wbuf1, rows1, gsem1, ssem1, stsem1)]

    def fire_stage_r(c, par):
        ridx, _, _, _, _, _, stsem = bufs[par]
        eoff = pl.multiple_of((chbase + c) * KE, 8)
        pltpu.async_copy(r_hbm.at[pl.ds(eoff, KE)], ridx, stsem)

    def fire_stage_cw(c, par):
        _, cidx, wbuf, _, _, _, stsem = bufs[par]
        ch = chbase + c
        eoff = pl.multiple_of(ch * KE, 8)
        pltpu.async_copy(c2d.at[pl.ds(ch * K, K)], cidx, stsem)
        pltpu.async_copy(w_hbm.at[pl.ds(eoff, KE)], wbuf, stsem)

    def drain_stage(c, par):
        ridx, cidx, wbuf, _, _, _, stsem = bufs[par]
        ch = chbase + c
        eoff = pl.multiple_of(ch * KE, 8)
        pltpu.make_async_copy(r_hbm.at[pl.ds(eoff, KE)], ridx, stsem).wait()
        pltpu.make_async_copy(c2d.at[pl.ds(ch * K, K)], cidx, stsem).wait()
        pltpu.make_async_copy(w_hbm.at[pl.ds(eoff, KE)], wbuf, stsem).wait()

    def shift_fire_gathers(par):
        ridx, _, _, rows, gsem, _, _ = bufs[par]
        if feat_split:
            @plsc.parallel_loop(0, KE // 16, unroll=4)
            def _shift(i):
                ridx[pl.ds(i * 16, 16)] = ridx[pl.ds(i * 16, 16)] + idx_off

        def fire(j, _):
            pltpu.async_copy(
                p_hbm.at[ridx.at[pl.ds(j * BLK, BLK)]], rows.at[j], gsem)
            return 0
        lax.fori_loop(0, K, fire, 0)

    def drain_gathers(par):
        ridx, _, _, rows, gsem, _, _ = bufs[par]

        def drain(j, _):
            pltpu.make_async_copy(
                p_hbm.at[ridx.at[pl.ds(j * BLK, BLK)]], rows.at[j],
                gsem).wait()
            return 0
        lax.fori_loop(0, K, drain, 0)

    def scale_scatter(par):
        _, cidx, wbuf, rows, _, ssem, _ = bufs[par]

        def scale_blk(j, _):
            jj = jnp.zeros((16,), _i32) + j
            wbase = j * BLK

            @plsc.parallel_loop(0, BLK // 2, unroll=4)
            def _scale(l2):
                ir = rowpat + 2 * l2
                v = plsc.load_gather(rows, [jj, ir, colpat])
                wv = plsc.load_gather(wbuf, [wbase + ir])
                plsc.store_scatter(rows, [jj, ir, colpat], v * wv)

            pltpu.async_copy(rows.at[j], acc.at[cidx.at[j]], ssem, add=True)
            return 0
        lax.fori_loop(0, K, scale_blk, 0)

        def sdrain(j, _):
            pltpu.make_async_copy(rows.at[j], acc.at[cidx.at[j]], ssem).wait()
            return 0
        lax.fori_loop(0, K, sdrain, 0)

    fire_stage_r(0, 0)
    fire_stage_cw(0, 0)
    fire_stage_r(1, 1)
    fire_stage_cw(1, 1)
    drain_stage(0, 0)
    shift_fire_gathers(0)

    def pair(b2, _):
        drain_gathers(0)
        drain_stage(2 * b2 + 1, 1)
        shift_fire_gathers(1)

        @pl.when(b2 + 1 < nchh)
        def _():
            fire_stage_r(2 * b2 + 2, 0)
        scale_scatter(0)

        @pl.when(b2 + 1 < nchh)
        def _():
            fire_stage_cw(2 * b2 + 2, 0)

        drain_gathers(1)

        @pl.when(b2 + 1 < nchh)
        def _():
            drain_stage(2 * b2 + 2, 0)
            shift_fire_gathers(0)
            fire_stage_r(2 * b2 + 3, 1)
        scale_scatter(1)

        @pl.when(b2 + 1 < nchh)
        def _():
            fire_stage_cw(2 * b2 + 3, 1)
        return 0
    lax.fori_loop(0, nchh, pair, 0)

    plsc.subcore_barrier()
    pltpu.sync_copy(acc.at[pl.ds(sid * RPT, RPT)],
                    out_hbm.at[cid, pl.ds(sid * RPT, RPT)])


def _agg_call(feat_split, r, c2d, w, p):
    mesh = plsc.VectorSubcoreMesh(core_axis_name="c", subcore_axis_name="s")
    return pl.kernel(
        functools.partial(_agg_body, feat_split),
        out_type=jax.ShapeDtypeStruct((_NC, NP, 8), _f32),
        mesh=mesh,
        compiler_params=pltpu.CompilerParams(
            needs_layout_passes=False, use_tc_tiling_on_sc=False),
        scratch_types=[
            pltpu.VMEM((KE,), _i32),
            pltpu.VMEM((KE,), _i32),
            pltpu.VMEM((K, BLK), _i32),
            pltpu.VMEM((K, BLK), _i32),
            pltpu.VMEM((KE,), _f32),
            pltpu.VMEM((KE,), _f32),
            pltpu.VMEM((K, BLK, 8), _f32),
            pltpu.VMEM((K, BLK, 8), _f32),
            pltpu.VMEM((ZR, 8), _f32),
            pltpu.VMEM_SHARED((NP, 8), _f32),
            pltpu.SemaphoreType.DMA,
            pltpu.SemaphoreType.DMA,
            pltpu.SemaphoreType.DMA,
            pltpu.SemaphoreType.DMA,
            pltpu.SemaphoreType.DMA,
            pltpu.SemaphoreType.DMA,
        ],
    )(r, c2d, w, p)


_BN = 2048


def _dense1_body(degp_ref, x_ref, dinv_ref, p1_ref):
    deg = jnp.sum(degp_ref[...], axis=0) + 1.0
    dinv = lax.rsqrt(deg)
    dinv_ref[...] = dinv
    p1_ref[...] = x_ref[...] * dinv[:, None]


def _dense1_call(degp, x):
    grid = pl.cdiv(N_NODES, _BN)
    return pl.pallas_call(
        _dense1_body,
        grid=(grid,),
        in_specs=[
            pl.BlockSpec((_NW, _BN), lambda i: (0, i)),
            pl.BlockSpec((_BN, 8), lambda i: (i, 0)),
        ],
        out_specs=[
            pl.BlockSpec((_BN,), lambda i: (i,)),
            pl.BlockSpec((_BN, 8), lambda i: (i, 0)),
        ],
        out_shape=[
            jax.ShapeDtypeStruct((N_NODES,), _f32),
            jax.ShapeDtypeStruct((N_NODES, 8), _f32),
        ],
    )(degp, x)


def _dense2_body(a0_ref, a1_ref, p1_ref, dinv_ref, w_ref, b_ref,
                 p2lo_ref, p2hi_ref):
    dv = dinv_ref[...][:, None]
    t = dv * (a0_ref[...] + a1_ref[...] + p1_ref[...])
    z = jnp.dot(t, w_ref[...], preferred_element_type=_f32) + b_ref[...][None, :]
    p2 = dv * jnp.maximum(z, 0.0)
    p2lo_ref[...] = p2[:, :8]
    p2hi_ref[...] = p2[:, 8:]


def _dense2_call(a0, a1, p1, dinv, w, b):
    grid = pl.cdiv(N_NODES, _BN)
    return pl.pallas_call(
        _dense2_body,
        grid=(grid,),
        in_specs=[
            pl.BlockSpec((_BN, 8), lambda i: (i, 0)),
            pl.BlockSpec((_BN, 8), lambda i: (i, 0)),
            pl.BlockSpec((_BN, 8), lambda i: (i, 0)),
            pl.BlockSpec((_BN,), lambda i: (i,)),
            pl.BlockSpec((8, 16), lambda i: (0, 0)),
            pl.BlockSpec((16,), lambda i: (0,)),
        ],
        out_specs=[
            pl.BlockSpec((_BN, 8), lambda i: (i, 0)),
            pl.BlockSpec((_BN, 8), lambda i: (i, 0)),
        ],
        out_shape=[
            jax.ShapeDtypeStruct((N_NODES, 8), _f32),
            jax.ShapeDtypeStruct((N_NODES, 8), _f32),
        ],
    )(a0, a1, p1, dinv, w, b)


def _dense3_body(alo_ref, ahi_ref, plo_ref, phi_ref, dinv_ref, w_ref, b_ref,
                 h_ref):
    dv = dinv_ref[...][:, None]
    t = dv * jnp.concatenate(
        [alo_ref[...] + plo_ref[...], ahi_ref[...] + phi_ref[...]], axis=1)
    z = jnp.dot(t, w_ref[...], preferred_element_type=_f32) + b_ref[...][None, :]
    h_ref[...] = jnp.maximum(z, 0.0)


def _dense3_call(alo, ahi, plo, phi, dinv, w, b):
    grid = pl.cdiv(N_NODES, _BN)
    return pl.pallas_call(
        _dense3_body,
        grid=(grid,),
        in_specs=[
            pl.BlockSpec((_BN, 8), lambda i: (i, 0)),
            pl.BlockSpec((_BN, 8), lambda i: (i, 0)),
            pl.BlockSpec((_BN, 8), lambda i: (i, 0)),
            pl.BlockSpec((_BN, 8), lambda i: (i, 0)),
            pl.BlockSpec((_BN,), lambda i: (i,)),
            pl.BlockSpec((16, 32), lambda i: (0, 0)),
            pl.BlockSpec((32,), lambda i: (0,)),
        ],
        out_specs=pl.BlockSpec((_BN, 32), lambda i: (i, 0)),
        out_shape=jax.ShapeDtypeStruct((N_NODES, 32), _f32),
    )(alo, ahi, plo, phi, dinv, w, b)


_BG = 512


def _fc_body(hr_ref, w_ref, b_ref, out_ref):
    logits = jnp.dot(hr_ref[...], w_ref[...], preferred_element_type=_f32)
    logits = logits + b_ref[...][None, :]
    lane = lax.broadcasted_iota(_i32, logits.shape, 1)
    neg = jnp.where(lane < 5, logits, -1e30)
    m = jnp.max(neg, axis=1, keepdims=True)
    lse = jnp.log(jnp.sum(jnp.exp(neg - m), axis=1, keepdims=True)) + m
    out_ref[...] = logits - lse


def _fc_call(hr, wf, bf):
    grid = pl.cdiv(NUM_GRAPHS, _BG)
    return pl.pallas_call(
        _fc_body,
        grid=(grid,),
        in_specs=[
            pl.BlockSpec((_BG, 256), lambda i: (i, 0)),
            pl.BlockSpec((256, 128), lambda i: (0, 0)),
            pl.BlockSpec((128,), lambda i: (0,)),
        ],
        out_specs=pl.BlockSpec((_BG, 128), lambda i: (i, 0)),
        out_shape=jax.ShapeDtypeStruct((NUM_GRAPHS, 128), _f32),
    )(hr, wf, bf)


def kernel(x, edge_index, edge_weight, W1, b1, W2, b2, Wfc, bfc):
    r = edge_index[0].astype(_i32)
    c = edge_index[1].astype(_i32)
    w = edge_weight.astype(_f32)
    c2d = c.reshape(NBLK, BLK)

    degp = _deg_call(c, w).reshape(_NW, NP)
    dinv, p1 = _dense1_call(degp, x)

    agg1 = _agg_call(False, r, c2d, w, p1)
    p2lo, p2hi = _dense2_call(agg1[0], agg1[1], p1, dinv, W1, b1)

    p2s = jnp.concatenate([p2lo, p2hi], axis=0)
    agg2 = _agg_call(True, r, c2d, w, p2s)
    h2 = _dense3_call(agg2[0], agg2[1], p2lo, p2hi, dinv, W2, b2)

    hr = h2.reshape(NUM_GRAPHS, 256)
    wf = jnp.pad(Wfc, ((0, 0), (0, 128 - Wfc.shape[1])))
    bf = jnp.pad(bfc, ((0, 128 - bfc.shape[0]),))
    out = _fc_call(hr, wf, bf)
    return out[:, :5]
